# Initial kernel scaffold; baseline (speedup 1.0000x reference)
import jax
import jax.numpy as jnp
from jax.experimental import pallas as pl


def kernel(x, gt_label, params0, params1, params2):
    raise NotImplementedError("write your pallas kernel here")


# speedup vs baseline: 4.1298x; 4.1298x over previous
import functools

import jax
import jax.numpy as jnp
import numpy as np
from jax import lax
from jax.experimental import pallas as pl
from jax.experimental.pallas import tpu as pltpu

_SCALE = 2


def _conv3x3(x_hwc, w, b):
    h, wd, ci = x_hwc.shape
    xp = jnp.pad(x_hwc, ((1, 1), (1, 1), (0, 0)))
    cols = [xp[dh:dh + h, dw:dw + wd, :] for dh in range(3) for dw in range(3)]
    xc = jnp.concatenate(cols, axis=-1).reshape(h * wd, 9 * ci)
    y = jnp.dot(xc, w, preferred_element_type=jnp.float32) + b[None, :]
    return y.reshape(h, wd, -1)


def _srnet_kernel(labels_ref, nblks_ref, x_ref, hw_ref, hb_ref, w1_ref, b1_ref,
                  w2_ref, b2_ref, bw_ref, bb_ref, uw_ref, ub_ref, tw_ref,
                  tb_ref, out_ref):
    i = pl.program_id(0)
    nblk = nblks_ref[i]

    x = x_ref[0]
    h0 = _conv3x3(x, hw_ref[0], hb_ref[0, 0])

    def block(j, r):
        t = jax.nn.relu(_conv3x3(r, w1_ref[0, j], b1_ref[0, j]))
        t = _conv3x3(t, w2_ref[0, j], b2_ref[0, j])
        return r + t

    r = lax.fori_loop(0, nblk, block, h0)
    r = _conv3x3(r, bw_ref[0], bb_ref[0, 0]) + h0
    u = _conv3x3(r, uw_ref[0], ub_ref[0, 0])
    u = u.reshape(32, 32, 2, 2, 64).transpose(0, 2, 1, 3, 4).reshape(64, 64, 64)
    out_ref[0] = _conv3x3(u, tw_ref[0], tb_ref[0, 0])


def _wmat(w):
    return jnp.transpose(w, (2, 3, 1, 0)).reshape(-1, w.shape[0])


@jax.jit
def kernel(x, gt_label, params0, params1, params2):
    experts = [params0, params1, params2]
    counts = [len(p['blocks']) for p in experts]
    maxb = max(counts)
    c = params0['head_w'].shape[0]

    hw = jnp.stack([_wmat(p['head_w']) for p in experts])
    hb = jnp.stack([p['head_b'] for p in experts])[:, None, :]

    def pad_blocks(p, field):
        mats = [_wmat(blk[field]) for blk in p['blocks']]
        mats += [jnp.zeros_like(mats[0])] * (maxb - len(mats))
        return jnp.stack(mats)

    def pad_bias(p, field):
        bs = [blk[field] for blk in p['blocks']]
        bs += [jnp.zeros_like(bs[0])] * (maxb - len(bs))
        return jnp.stack(bs)

    w1 = jnp.stack([pad_blocks(p, 'w1') for p in experts])
    b1 = jnp.stack([pad_bias(p, 'b1') for p in experts])
    w2 = jnp.stack([pad_blocks(p, 'w2') for p in experts])
    b2 = jnp.stack([pad_bias(p, 'b2') for p in experts])
    bw = jnp.stack([_wmat(p['body_w']) for p in experts])
    bb = jnp.stack([p['body_b'] for p in experts])[:, None, :]

    r2 = _SCALE * _SCALE
    perm = np.empty(c * r2, dtype=np.int32)
    for rh in range(_SCALE):
        for rw in range(_SCALE):
            for co in range(c):
                perm[(rh * _SCALE + rw) * c + co] = co * r2 + rh * _SCALE + rw
    uw = jnp.stack([_wmat(p['up_w'][perm]) for p in experts])
    ub = jnp.stack([p['up_b'][perm] for p in experts])[:, None, :]
    tw = jnp.stack([_wmat(p['tail_w']) for p in experts])
    tb = jnp.stack([p['tail_b'] for p in experts])[:, None, :]

    b = x.shape[0]
    xh = jnp.transpose(x, (0, 2, 3, 1))
    labels = gt_label.astype(jnp.int32)
    nblks = jnp.asarray(counts, jnp.int32)[labels]

    def exp_map(i, lbl, nb):
        return (lbl[i], 0, 0)

    def exp_map4(i, lbl, nb):
        return (lbl[i], 0, 0, 0)

    grid_spec = pltpu.PrefetchScalarGridSpec(
        num_scalar_prefetch=2,
        grid=(b,),
        in_specs=[
            pl.BlockSpec((1, 32, 32, 3), lambda i, lbl, nb: (i, 0, 0, 0)),
            pl.BlockSpec((1,) + hw.shape[1:], exp_map),
            pl.BlockSpec((1,) + hb.shape[1:], exp_map),
            pl.BlockSpec((1,) + w1.shape[1:], exp_map4),
            pl.BlockSpec((1,) + b1.shape[1:], exp_map),
            pl.BlockSpec((1,) + w2.shape[1:], exp_map4),
            pl.BlockSpec((1,) + b2.shape[1:], exp_map),
            pl.BlockSpec((1,) + bw.shape[1:], exp_map),
            pl.BlockSpec((1,) + bb.shape[1:], exp_map),
            pl.BlockSpec((1,) + uw.shape[1:], exp_map),
            pl.BlockSpec((1,) + ub.shape[1:], exp_map),
            pl.BlockSpec((1,) + tw.shape[1:], exp_map),
            pl.BlockSpec((1,) + tb.shape[1:], exp_map),
        ],
        out_specs=pl.BlockSpec((1, 64, 64, 3), lambda i, lbl, nb: (i, 0, 0, 0)),
    )

    out = pl.pallas_call(
        _srnet_kernel,
        grid_spec=grid_spec,
        out_shape=jax.ShapeDtypeStruct((b, 64, 64, 3), jnp.float32),
        compiler_params=pltpu.CompilerParams(
            vmem_limit_bytes=100 * 1024 * 1024,
        ),
    )(labels, nblks, xh, hw, hb, w1, b1, w2, b2, bw, bb, uw, ub, tw, tb)

    return jnp.transpose(out, (0, 3, 1, 2))
